# Initial kernel scaffold; baseline (speedup 1.0000x reference)
import jax
import jax.numpy as jnp
from jax.experimental import pallas as pl


def kernel(x, batch, ei, W):
    raise NotImplementedError("write your pallas kernel here")


# speedup vs baseline: 9.5813x; 9.5813x over previous
import functools

import jax
import jax.numpy as jnp
from jax import lax
from jax.experimental import pallas as pl
from jax.experimental.pallas import tpu as pltpu
from jax.experimental.pallas import tpu_sc as plsc

_FIN = 128
_FQK = 64
_N = 10000
_E = 320000
_NPAD = 10240
_NC, _NS, _L = 2, 16, 16
_NW = _NC * _NS
_CH = 128
_NCHUNK = _E // _CH
_BASE_CNT = _NCHUNK // _NW
_EXTRA = _NCHUNK - _BASE_CNT * _NW
_NODES_PER_TILE = _NPAD // _NS
_GROUPS = _CH // _L


def _proj_body(x_ref, w_ref, q_ref, k_ref):
    qk = jnp.dot(x_ref[...], w_ref[...], preferred_element_type=jnp.float32)
    scale = float(_FQK) ** (-0.5)
    q_ref[...] = qk[:, :_FQK] * scale
    k_ref[...] = qk[:, _FQK:]


def _project(x, W):
    return pl.pallas_call(
        _proj_body,
        out_shape=(
            jax.ShapeDtypeStruct((_N, _FQK), jnp.float32),
            jax.ShapeDtypeStruct((_N, _FQK), jnp.float32),
        ),
    )(x, W)


def _worker_span(wid):
    base = wid * _BASE_CNT + jnp.minimum(wid, _EXTRA)
    cnt = _BASE_CNT + jnp.where(wid < _EXTRA, 1, 0)
    return base, cnt


def _zero_ref(ref, nwords):
    zeros = jnp.zeros((_L,), jnp.float32)

    def body(i, _):
        ref[pl.ds(i * _L, _L)] = zeros
        return 0

    lax.fori_loop(0, nwords // _L, body, 0)


def _edge_body(q_hbm, k_hbm, src_hbm, dest_hbm,
               exp_hbm, p0_hbm, p1_hbm,
               sidx_v, didx_v, qrows_v, krows_v, exp_v, acc_v,
               tmp_v, tot_v, shared_sp, sem):
    cid = lax.axis_index("c")
    sid = lax.axis_index("s")
    wid = sid * _NC + cid
    base, cnt = _worker_span(wid)

    _zero_ref(acc_v, _NPAD)

    def chunk_body(i, _):
        ebase = (base + i) * _CH
        pltpu.sync_copy(src_hbm.at[pl.ds(ebase, _CH)], sidx_v)
        pltpu.sync_copy(dest_hbm.at[pl.ds(ebase, _CH)], didx_v)
        qd = pltpu.async_copy(q_hbm.at[sidx_v], qrows_v, sem)
        kd = pltpu.async_copy(k_hbm.at[didx_v], krows_v, sem)
        qd.wait()
        kd.wait()
        lane = jnp.arange(_L, dtype=jnp.int32)
        for g in range(_GROUPS):
            dots = jnp.zeros((_L,), jnp.float32)
            for e in range(_L):
                row = g * _L + e
                prod = jnp.zeros((_L,), jnp.float32)
                for j in range(_FQK // _L):
                    sl = pl.ds(j * _L, _L)
                    prod = prod + qrows_v[row, sl] * krows_v[row, sl]
                dots = jnp.where(lane == e, jnp.sum(prod), dots)
            ev = jnp.exp(dots)
            sl = pl.ds(g * _L, _L)
            exp_v[sl] = ev
            srcv = sidx_v[sl]
            plsc.addupdate_scatter(acc_v, [srcv], ev)
        pltpu.sync_copy(exp_v, exp_hbm.at[pl.ds(ebase, _CH)])
        return 0

    lax.fori_loop(0, cnt, chunk_body, 0)

    pltpu.sync_copy(acc_v, shared_sp.at[sid])
    plsc.subcore_barrier()

    nbase = sid * _NODES_PER_TILE
    _zero_ref(tot_v, _NODES_PER_TILE)
    for r in range(_NS):
        pltpu.sync_copy(shared_sp.at[r, pl.ds(nbase, _NODES_PER_TILE)], tmp_v)

        def add_body(j, _):
            sl = pl.ds(j * _L, _L)
            tot_v[sl] = tot_v[sl] + tmp_v[sl]
            return 0

        lax.fori_loop(0, _NODES_PER_TILE // _L, add_body, 0)

    @pl.when(cid == 0)
    def _():
        pltpu.sync_copy(tot_v, p0_hbm.at[pl.ds(nbase, _NODES_PER_TILE)])

    @pl.when(cid == 1)
    def _():
        pltpu.sync_copy(tot_v, p1_hbm.at[pl.ds(nbase, _NODES_PER_TILE)])


def _edge_kernel(q, k, src, dest):
    mesh = plsc.VectorSubcoreMesh(core_axis_name="c", subcore_axis_name="s")
    kfn = pl.kernel(
        _edge_body,
        out_type=(
            jax.ShapeDtypeStruct((_E,), jnp.float32),
            jax.ShapeDtypeStruct((_NPAD,), jnp.float32),
            jax.ShapeDtypeStruct((_NPAD,), jnp.float32),
        ),
        mesh=mesh,
        compiler_params=pltpu.CompilerParams(
            needs_layout_passes=False, use_tc_tiling_on_sc=False),
        scratch_types=(
            pltpu.VMEM((_CH,), jnp.int32),
            pltpu.VMEM((_CH,), jnp.int32),
            pltpu.VMEM((_CH, _FQK), jnp.float32),
            pltpu.VMEM((_CH, _FQK), jnp.float32),
            pltpu.VMEM((_CH,), jnp.float32),
            pltpu.VMEM((_NPAD,), jnp.float32),
            pltpu.VMEM((_NODES_PER_TILE,), jnp.float32),
            pltpu.VMEM((_NODES_PER_TILE,), jnp.float32),
            pltpu.VMEM_SHARED((_NS, _NPAD), jnp.float32),
            pltpu.SemaphoreType.DMA,
        ),
    )
    return kfn(q, k, src, dest)


def _norm_body(exp_hbm, src_hbm, p0_hbm, p1_hbm, out_hbm,
               sum_v, tmp_v, sidx_v, ev_v, out_v):
    cid = lax.axis_index("c")
    sid = lax.axis_index("s")
    wid = sid * _NC + cid
    base, cnt = _worker_span(wid)

    pltpu.sync_copy(p0_hbm, sum_v)
    pltpu.sync_copy(p1_hbm, tmp_v)

    def add_body(j, _):
        sl = pl.ds(j * _L, _L)
        sum_v[sl] = sum_v[sl] + tmp_v[sl]
        return 0

    lax.fori_loop(0, _NPAD // _L, add_body, 0)

    def chunk_body(i, _):
        ebase = (base + i) * _CH
        pltpu.sync_copy(src_hbm.at[pl.ds(ebase, _CH)], sidx_v)
        pltpu.sync_copy(exp_hbm.at[pl.ds(ebase, _CH)], ev_v)
        for g in range(_GROUPS):
            sl = pl.ds(g * _L, _L)
            srcv = sidx_v[sl]
            sv = plsc.load_gather(sum_v, [srcv])
            out_v[sl] = ev_v[sl] / sv
        pltpu.sync_copy(out_v, out_hbm.at[pl.ds(ebase, _CH)])
        return 0

    lax.fori_loop(0, cnt, chunk_body, 0)


def _normalize(exp_aw, src, p0, p1):
    mesh = plsc.VectorSubcoreMesh(core_axis_name="c", subcore_axis_name="s")
    kfn = pl.kernel(
        _norm_body,
        out_type=jax.ShapeDtypeStruct((_E,), jnp.float32),
        mesh=mesh,
        compiler_params=pltpu.CompilerParams(
            needs_layout_passes=False, use_tc_tiling_on_sc=False),
        scratch_types=(
            pltpu.VMEM((_NPAD,), jnp.float32),
            pltpu.VMEM((_NPAD,), jnp.float32),
            pltpu.VMEM((_CH,), jnp.int32),
            pltpu.VMEM((_CH,), jnp.float32),
            pltpu.VMEM((_CH,), jnp.float32),
        ),
    )
    return kfn(exp_aw, src, p0, p1)


def kernel(x, batch, ei, W):
    del batch
    src = ei[0]
    dest = ei[1]
    q, k = _project(x, W)
    exp_aw, p0, p1 = _edge_kernel(q, k, src, dest)
    return _normalize(exp_aw, src, p0, p1)


# speedup vs baseline: 22.1683x; 2.3137x over previous
import jax
import jax.numpy as jnp
from jax import lax
from jax.experimental import pallas as pl
from jax.experimental.pallas import tpu as pltpu
from jax.experimental.pallas import tpu_sc as plsc

_FIN = 128
_FQK = 64
_N = 10000
_E = 320000
_NPAD = 10240
_NC, _NS, _L = 2, 16, 16
_NW = _NC * _NS
_CH = 128
_NCHUNK = _E // _CH
_BASE_CNT = _NCHUNK // _NW
_EXTRA = _NCHUNK - _BASE_CNT * _NW
_LOOP_CH = 80
_SPAN = _LOOP_CH * _CH
_EPAD = 320512
_NODES_PER_TILE = _NPAD // _NS
_GROUPS = _CH // _L


def _proj_body(x_ref, w_ref, q_ref, k_ref):
    qk = jnp.dot(x_ref[...], w_ref[...], preferred_element_type=jnp.float32)
    scale = float(_FQK) ** (-0.5)
    q_ref[...] = qk[:, :_FQK] * scale
    k_ref[...] = qk[:, _FQK:]


def _project(x, W):
    return pl.pallas_call(
        _proj_body,
        out_shape=(
            jax.ShapeDtypeStruct((_N, _FQK), jnp.float32),
            jax.ShapeDtypeStruct((_N, _FQK), jnp.float32),
        ),
    )(x, W)


def _worker_span(wid):
    base = wid * _BASE_CNT + jnp.minimum(wid, _EXTRA)
    cnt = _BASE_CNT + jnp.where(wid < _EXTRA, 1, 0)
    return base, cnt


def _zero_ref(ref, nwords):
    zeros = jnp.zeros((_L,), jnp.float32)

    def body(i, _):
        ref[pl.ds(i * _L, _L)] = zeros
        return 0

    lax.fori_loop(0, nwords // _L, body, 0)


_SC_PARAMS = pltpu.CompilerParams(
    needs_layout_passes=False, use_tc_tiling_on_sc=False)


def _edge_body(q_hbm, k_hbm, src_hbm, dest_hbm,
               exp_hbm, p0_hbm, p1_hbm,
               sidx_v, didx_v, qr0_v, kr0_v, qr1_v, kr1_v,
               expall_v, acc_v, tmp_v, tot_v, shared_sp, sem0, sem1):
    cid = lax.axis_index("c")
    sid = lax.axis_index("s")
    wid = sid * _NC + cid
    base, cnt = _worker_span(wid)
    e0 = base * _CH

    pltpu.sync_copy(src_hbm.at[pl.ds(e0, _SPAN)], sidx_v)
    pltpu.sync_copy(dest_hbm.at[pl.ds(e0, _SPAN)], didx_v)
    _zero_ref(acc_v, _NPAD)

    bufs = ((qr0_v, kr0_v, sem0), (qr1_v, kr1_v, sem1))

    def _gather(c, p):
        qr, kr, sem = bufs[p]
        pltpu.async_copy(q_hbm.at[sidx_v.at[pl.ds(c * _CH, _CH)]], qr, sem)
        pltpu.async_copy(k_hbm.at[didx_v.at[pl.ds(c * _CH, _CH)]], kr, sem)

    _gather(0, 0)
    lane = jnp.arange(_L, dtype=jnp.int32)

    def pair_body(gi, _):
        for p in range(2):
            c = gi * 2 + p
            qr, kr, sem = bufs[p]

            @pl.when(c + 1 < _LOOP_CH)
            def _():
                _gather(c + 1, 1 - p)

            pltpu.make_async_copy(
                q_hbm.at[sidx_v.at[pl.ds(c * _CH, _CH)]], qr, sem).wait()
            pltpu.make_async_copy(
                k_hbm.at[didx_v.at[pl.ds(c * _CH, _CH)]], kr, sem).wait()

            in_range = c < cnt
            smask = jnp.full((_L,), in_range)
            lax.fori_loop(0, _GROUPS, _rowwise_groups(qr, kr, sidx_v, expall_v,
                                                      acc_v, smask, lane, c), 0)
        return 0

    lax.fori_loop(0, _LOOP_CH // 2, pair_body, 0)

    main_words = _BASE_CNT * _CH
    pltpu.sync_copy(expall_v.at[pl.ds(0, main_words)],
                    exp_hbm.at[pl.ds(e0, main_words)])

    @pl.when(cnt == _BASE_CNT + 1)
    def _():
        pltpu.sync_copy(expall_v.at[pl.ds(main_words, _CH)],
                        exp_hbm.at[pl.ds(e0 + main_words, _CH)])

    pltpu.sync_copy(acc_v, shared_sp.at[sid])
    plsc.subcore_barrier()

    nbase = sid * _NODES_PER_TILE
    _zero_ref(tot_v, _NODES_PER_TILE)
    for r in range(_NS):
        pltpu.sync_copy(shared_sp.at[r, pl.ds(nbase, _NODES_PER_TILE)], tmp_v)

        def add_body(j, _):
            sl = pl.ds(j * _L, _L)
            tot_v[sl] = tot_v[sl] + tmp_v[sl]
            return 0

        lax.fori_loop(0, _NODES_PER_TILE // _L, add_body, 0)

    @pl.when(cid == 0)
    def _():
        pltpu.sync_copy(tot_v, p0_hbm.at[pl.ds(nbase, _NODES_PER_TILE)])

    @pl.when(cid == 1)
    def _():
        pltpu.sync_copy(tot_v, p1_hbm.at[pl.ds(nbase, _NODES_PER_TILE)])


def _rowwise_groups(qr, kr, sidx_v, expall_v, acc_v, smask, lane, c):
    def group_body(g, carry):
        dots = jnp.zeros((_L,), jnp.float32)
        for e in range(_L):
            prod = jnp.zeros((_L,), jnp.float32)
            row = g * _L + e
            for j in range(_FQK // _L):
                sl = pl.ds(j * _L, _L)
                prod = prod + qr[row, sl] * kr[row, sl]
            dots = jnp.where(lane == e, jnp.sum(prod), dots)
        ev = jnp.exp(dots)
        off = c * _CH + g * _L
        expall_v[pl.ds(off, _L)] = ev
        srcv = sidx_v[pl.ds(off, _L)]
        plsc.addupdate_scatter(acc_v, [srcv], ev, mask=smask)
        return carry

    return group_body


def _edge_kernel(q, k, src_pad, dest_pad):
    mesh = plsc.VectorSubcoreMesh(core_axis_name="c", subcore_axis_name="s")
    kfn = pl.kernel(
        _edge_body,
        out_type=(
            jax.ShapeDtypeStruct((_E,), jnp.float32),
            jax.ShapeDtypeStruct((_NPAD,), jnp.float32),
            jax.ShapeDtypeStruct((_NPAD,), jnp.float32),
        ),
        mesh=mesh,
        compiler_params=_SC_PARAMS,
        scratch_types=(
            pltpu.VMEM((_SPAN,), jnp.int32),
            pltpu.VMEM((_SPAN,), jnp.int32),
            pltpu.VMEM((_CH, _FQK), jnp.float32),
            pltpu.VMEM((_CH, _FQK), jnp.float32),
            pltpu.VMEM((_CH, _FQK), jnp.float32),
            pltpu.VMEM((_CH, _FQK), jnp.float32),
            pltpu.VMEM((_SPAN,), jnp.float32),
            pltpu.VMEM((_NPAD,), jnp.float32),
            pltpu.VMEM((_NODES_PER_TILE,), jnp.float32),
            pltpu.VMEM((_NODES_PER_TILE,), jnp.float32),
            pltpu.VMEM_SHARED((_NS, _NPAD), jnp.float32),
            pltpu.SemaphoreType.DMA,
            pltpu.SemaphoreType.DMA,
        ),
    )
    return kfn(q, k, src_pad, dest_pad)


def _norm_body(exp_hbm, src_hbm, p0_hbm, p1_hbm, out_hbm,
               sum_v, tmp_v, sidx_v, eall_v, oall_v):
    cid = lax.axis_index("c")
    sid = lax.axis_index("s")
    wid = sid * _NC + cid
    base, cnt = _worker_span(wid)
    e0 = base * _CH
    main_words = _BASE_CNT * _CH

    pltpu.sync_copy(p0_hbm, sum_v)
    pltpu.sync_copy(p1_hbm, tmp_v)
    pltpu.sync_copy(src_hbm.at[pl.ds(e0, _SPAN)], sidx_v)
    pltpu.sync_copy(exp_hbm.at[pl.ds(e0, main_words)],
                    eall_v.at[pl.ds(0, main_words)])

    @pl.when(cnt == _BASE_CNT + 1)
    def _():
        pltpu.sync_copy(exp_hbm.at[pl.ds(e0 + main_words, _CH)],
                        eall_v.at[pl.ds(main_words, _CH)])

    def add_body(j, _):
        sl = pl.ds(j * _L, _L)
        sum_v[sl] = sum_v[sl] + tmp_v[sl]
        return 0

    lax.fori_loop(0, _NPAD // _L, add_body, 0)

    def group_body(g, _):
        sl = pl.ds(g * _L, _L)
        srcv = sidx_v[sl]
        sv = plsc.load_gather(sum_v, [srcv])
        oall_v[sl] = eall_v[sl] / sv
        return 0

    lax.fori_loop(0, cnt * _GROUPS, group_body, 0)

    pltpu.sync_copy(oall_v.at[pl.ds(0, main_words)],
                    out_hbm.at[pl.ds(e0, main_words)])

    @pl.when(cnt == _BASE_CNT + 1)
    def _():
        pltpu.sync_copy(oall_v.at[pl.ds(main_words, _CH)],
                        out_hbm.at[pl.ds(e0 + main_words, _CH)])


def _normalize(exp_aw, src_pad, p0, p1):
    mesh = plsc.VectorSubcoreMesh(core_axis_name="c", subcore_axis_name="s")
    kfn = pl.kernel(
        _norm_body,
        out_type=jax.ShapeDtypeStruct((_E,), jnp.float32),
        mesh=mesh,
        compiler_params=_SC_PARAMS,
        scratch_types=(
            pltpu.VMEM((_NPAD,), jnp.float32),
            pltpu.VMEM((_NPAD,), jnp.float32),
            pltpu.VMEM((_SPAN,), jnp.int32),
            pltpu.VMEM((_SPAN,), jnp.float32),
            pltpu.VMEM((_SPAN,), jnp.float32),
        ),
    )
    return kfn(exp_aw, src_pad, p0, p1)


def kernel(x, batch, ei, W):
    del batch
    pad = jnp.zeros((_EPAD - _E,), jnp.int32)
    src_pad = jnp.concatenate([ei[0], pad])
    dest_pad = jnp.concatenate([ei[1], pad])
    q, k = _project(x, W)
    exp_aw, p0, p1 = _edge_kernel(q, k, src_pad, dest_pad)
    return _normalize(exp_aw, src_pad, p0, p1)
